# baseline (speedup 1.0000x reference)
import functools

import jax
import jax.numpy as jnp
import numpy as np
from jax import lax
from jax.experimental import pallas as pl
from jax.experimental.pallas import tpu as pltpu
from jax.experimental.pallas import tpu_sc as plsc

_N_ROWS = 100000
_D = 256
_N_SEG = 512
_NC = 2
_NS = 16
_CH = _D // _NC
_LANE = 16
_CHUNK = 128
_OWN = (-(-_N_ROWS // _NS) + 7) // 8 * 8
_NCHUNK = -(-_OWN // _CHUNK)
_WIN = _NCHUNK * _CHUNK
_ACC_ROWS = _N_SEG + 8
_SEG_PER_TILE = _N_SEG // _NS
_PHASES = 4
_SEG_PER_PHASE = _N_SEG // _PHASES
_SEG_PER_PT = _SEG_PER_PHASE // _NS

_W_OFF = np.minimum(np.arange(_NS) * _OWN, _N_ROWS - _WIN)
_ROW_IDX = _W_OFF[:, None] + np.arange(_WIN)[None, :]
_REAL_LO = np.arange(_NS) * _OWN
_REAL_HI = np.append(_REAL_LO[1:], _N_ROWS)
_OWN_MASK = (_ROW_IDX >= _REAL_LO[:, None]) & (_ROW_IDX < _REAL_HI[:, None])


def _pool_body(
    feat_hbm, ids_hbm, out_hbm, ids_v, buf_v, acc_v, cbuf_v, csum_v,
    stage_sh, sem0, sem1
):
    c = lax.axis_index("c")
    s = lax.axis_index("s")
    w_off = lax.min(s * _OWN, _N_ROWS - _WIN)
    col0 = c * _CH
    zero16 = jnp.zeros((_LANE,), jnp.float32)

    def _zero_acc(i, carry):
        acc_v[i // 8, pl.ds((i % 8) * _LANE, _LANE)] = zero16
        return carry

    lax.fori_loop(0, _ACC_ROWS * (_CH // _LANE), _zero_acc, 0)

    pltpu.sync_copy(ids_hbm.at[s], ids_v)

    def _src(j):
        row0 = pl.multiple_of(w_off + j * _CHUNK, 8)
        return feat_hbm.at[pl.ds(row0, _CHUNK), pl.ds(col0, _CH)]

    def _accumulate(j, buf):
        def _group(g, carry):
            idsvec = ids_v[j, pl.ds(g * _LANE, _LANE)]
            for l in range(_LANE):
                seg = idsvec[l]
                r = g * _LANE + l
                for k in range(_CH // _LANE):
                    plsc.addupdate(
                        acc_v.at[seg, pl.ds(k * _LANE, _LANE)],
                        buf[r, pl.ds(k * _LANE, _LANE)],
                    )
            return carry

        lax.fori_loop(0, _CHUNK // _LANE, _group, 0)

    pltpu.async_copy(_src(0), buf_v.at[0], sem0)

    def _pair(t, carry):
        j0 = 2 * t
        pltpu.async_copy(_src(j0 + 1), buf_v.at[1], sem1)
        pltpu.make_async_copy(_src(j0), buf_v.at[0], sem0).wait()
        _accumulate(j0, buf_v.at[0])
        pltpu.async_copy(_src(j0 + 2), buf_v.at[0], sem0)
        pltpu.make_async_copy(_src(j0 + 1), buf_v.at[1], sem1).wait()
        _accumulate(j0 + 1, buf_v.at[1])
        return carry

    lax.fori_loop(0, (_NCHUNK - 1) // 2, _pair, 0)
    pltpu.make_async_copy(_src(_NCHUNK - 1), buf_v.at[0], sem0).wait()
    _accumulate(_NCHUNK - 1, buf_v.at[0])

    for p in range(_PHASES):
        if p:
            plsc.subcore_barrier()
        pltpu.sync_copy(
            acc_v.at[pl.ds(p * _SEG_PER_PHASE, _SEG_PER_PHASE)],
            stage_sh.at[s],
        )
        plsc.subcore_barrier()

        def _zero_csum(i, carry):
            csum_v[i // 8, pl.ds((i % 8) * _LANE, _LANE)] = zero16
            return carry

        lax.fori_loop(0, _SEG_PER_PT * (_CH // _LANE), _zero_csum, 0)

        def _gather_tile(t, carry):
            pltpu.sync_copy(
                stage_sh.at[t, pl.ds(s * _SEG_PER_PT, _SEG_PER_PT)], cbuf_v
            )

            def _add(i, carry2):
                r = i // 8
                k = i % 8
                plsc.addupdate(
                    csum_v.at[r, pl.ds(k * _LANE, _LANE)],
                    cbuf_v[r, pl.ds(k * _LANE, _LANE)],
                )
                return carry2

            lax.fori_loop(0, _SEG_PER_PT * (_CH // _LANE), _add, 0)
            return carry

        lax.fori_loop(0, _NS, _gather_tile, 0)

        pltpu.sync_copy(
            csum_v,
            out_hbm.at[
                pl.ds(p * _SEG_PER_PHASE + s * _SEG_PER_PT, _SEG_PER_PT),
                pl.ds(col0, _CH),
            ],
        )


_pool = pl.kernel(
    _pool_body,
    out_type=jax.ShapeDtypeStruct((_N_SEG, _D), jnp.float32),
    mesh=plsc.VectorSubcoreMesh(core_axis_name="c", subcore_axis_name="s"),
    scratch_types=[
        pltpu.VMEM((_NCHUNK, _CHUNK), jnp.int32),
        pltpu.VMEM((2, _CHUNK, _CH), jnp.float32),
        pltpu.VMEM((_ACC_ROWS, _CH), jnp.float32),
        pltpu.VMEM((_SEG_PER_PT, _CH), jnp.float32),
        pltpu.VMEM((_SEG_PER_PT, _CH), jnp.float32),
        pltpu.VMEM_SHARED((_NS, _SEG_PER_PHASE, _CH), jnp.float32),
        pltpu.SemaphoreType.DMA,
        pltpu.SemaphoreType.DMA,
    ],
)


@jax.jit
def kernel(feat, segment_ids):
    ids = segment_ids.astype(jnp.int32)
    win = ids[jnp.asarray(_ROW_IDX)]
    ids_padded = jnp.where(jnp.asarray(_OWN_MASK), win, _N_SEG).reshape(
        _NS, _NCHUNK, _CHUNK
    )
    return _pool(feat, ids_padded)


# speedup vs baseline: 2.4756x; 2.4756x over previous
import functools

import jax
import jax.numpy as jnp
import numpy as np
from jax import lax
from jax.experimental import pallas as pl
from jax.experimental.pallas import tpu as pltpu
from jax.experimental.pallas import tpu_sc as plsc

_N_ROWS = 100000
_D = 256
_N_SEG = 512
_NC = 2
_NS = 16
_CH = _D // _NC
_LANE = 16
_CHUNK = 128
_OWN = (-(-_N_ROWS // _NS) + 7) // 8 * 8
_NCHUNK = -(-_OWN // _CHUNK)
_WIN = _NCHUNK * _CHUNK
_ACC_ROWS = 528
_ZROWS = _ACC_ROWS // _NS
_SEG_PER_TILE = _N_SEG // _NS

_W_OFF = np.minimum(np.arange(_NS) * _OWN, _N_ROWS - _WIN)
_ROW_IDX = _W_OFF[:, None] + np.arange(_WIN)[None, :]
_REAL_LO = np.arange(_NS) * _OWN
_REAL_HI = np.append(_REAL_LO[1:], _N_ROWS)
_OWN_MASK = (_ROW_IDX >= _REAL_LO[:, None]) & (_ROW_IDX < _REAL_HI[:, None])


def _pool_body(
    feat_hbm, ids_hbm, out_hbm, ids_v, buf_v, zrow_v, acc_sh, sem0, sem1, sem2
):
    c = lax.axis_index("c")
    s = lax.axis_index("s")
    w_off = lax.min(s * _OWN, _N_ROWS - _WIN)
    col0 = c * _CH
    zero16 = jnp.zeros((_LANE,), jnp.float32)

    def _zero(i, carry):
        zrow_v[i // 8, pl.ds((i % 8) * _LANE, _LANE)] = zero16
        return carry

    lax.fori_loop(0, _ZROWS * (_CH // _LANE), _zero, 0)
    pltpu.sync_copy(zrow_v, acc_sh.at[pl.ds(s * _ZROWS, _ZROWS)])

    pltpu.sync_copy(ids_hbm.at[s], ids_v)

    plsc.subcore_barrier()

    def _src(j):
        row0 = pl.multiple_of(w_off + j * _CHUNK, 8)
        return feat_hbm.at[pl.ds(row0, _CHUNK), pl.ds(col0, _CH)]

    sems = (sem0, sem1, sem2)

    pltpu.async_copy(_src(0), buf_v.at[0], sem0)
    pltpu.async_copy(_src(1), buf_v.at[1], sem1)

    def _slot(t, k):
        j = 3 * t + k
        pltpu.make_async_copy(_src(j), buf_v.at[k], sems[k]).wait()
        pltpu.sync_copy(buf_v.at[k], acc_sh.at[ids_v.at[j]], add=True)

        @pl.when(j + 2 < _NCHUNK)
        def _():
            kn = (k + 2) % 3
            pltpu.async_copy(_src(j + 2), buf_v.at[kn], sems[kn])

    def _trio(t, carry):
        _slot(t, 0)
        _slot(t, 1)
        _slot(t, 2)
        return carry

    lax.fori_loop(0, (_NCHUNK - 1) // 3, _trio, 0)
    _slot((_NCHUNK - 1) // 3, 0)

    plsc.subcore_barrier()
    pltpu.sync_copy(
        acc_sh.at[pl.ds(s * _SEG_PER_TILE, _SEG_PER_TILE)],
        out_hbm.at[
            pl.ds(s * _SEG_PER_TILE, _SEG_PER_TILE), pl.ds(col0, _CH)
        ],
    )


_pool = pl.kernel(
    _pool_body,
    out_type=jax.ShapeDtypeStruct((_N_SEG, _D), jnp.float32),
    mesh=plsc.VectorSubcoreMesh(core_axis_name="c", subcore_axis_name="s"),
    scratch_types=[
        pltpu.VMEM((_NCHUNK, _CHUNK), jnp.int32),
        pltpu.VMEM((3, _CHUNK, _CH), jnp.float32),
        pltpu.VMEM((_ZROWS, _CH), jnp.float32),
        pltpu.VMEM_SHARED((_ACC_ROWS, _CH), jnp.float32),
        pltpu.SemaphoreType.DMA,
        pltpu.SemaphoreType.DMA,
        pltpu.SemaphoreType.DMA,
    ],
)


@jax.jit
def kernel(feat, segment_ids):
    ids = segment_ids.astype(jnp.int32)
    win = jnp.stack(
        [lax.slice(ids, (int(o),), (int(o) + _WIN,)) for o in _W_OFF]
    )
    ids_padded = jnp.where(jnp.asarray(_OWN_MASK), win, _N_SEG).reshape(
        _NS, _NCHUNK, _CHUNK
    )
    return _pool(feat, ids_padded)


# speedup vs baseline: 2.8559x; 1.1536x over previous
import functools

import jax
import jax.numpy as jnp
import numpy as np
from jax import lax
from jax.experimental import pallas as pl
from jax.experimental.pallas import tpu as pltpu
from jax.experimental.pallas import tpu_sc as plsc

_N_ROWS = 100000
_D = 256
_N_SEG = 512
_NC = 2
_NS = 16
_CH = _D // _NC
_LANE = 16
_CHUNK = 128
_OWN = (-(-_N_ROWS // _NS) + 7) // 8 * 8
_NCHUNK = -(-_OWN // _CHUNK)
_WIN = _NCHUNK * _CHUNK
_ACC_ROWS = 528
_ZROWS = _ACC_ROWS // _NS
_SEG_PER_TILE = _N_SEG // _NS

_W_OFF = np.minimum(np.arange(_NS) * _OWN, _N_ROWS - _WIN)
_ROW_IDX = _W_OFF[:, None] + np.arange(_WIN)[None, :]
_REAL_LO = np.arange(_NS) * _OWN
_REAL_HI = np.append(_REAL_LO[1:], _N_ROWS)
_OWN_MASK = (_ROW_IDX >= _REAL_LO[:, None]) & (_ROW_IDX < _REAL_HI[:, None])


def _pool_body(
    feat_hbm, ids_hbm, out_hbm, ids_v, buf_v, zrow_v, mini_v, midx_v, acc_sh,
    sem0, sem1, sem2
):
    c = lax.axis_index("c")
    s = lax.axis_index("s")
    w_off = lax.min(s * _OWN, _N_ROWS - _WIN)
    col0 = c * _CH
    zero16 = jnp.zeros((_LANE,), jnp.float32)

    def _zero(i, carry):
        zrow_v[i // 8, pl.ds((i % 8) * _LANE, _LANE)] = zero16
        return carry

    lax.fori_loop(0, _ZROWS * (_CH // _LANE), _zero, 0)
    pltpu.sync_copy(zrow_v, acc_sh.at[pl.ds(s * _ZROWS, _ZROWS)])

    def _zero_mini(i, carry):
        mini_v[i // 8, pl.ds((i % 8) * _LANE, _LANE)] = zero16
        return carry

    lax.fori_loop(0, _LANE * (_CH // _LANE), _zero_mini, 0)

    pltpu.sync_copy(ids_hbm.at[s], ids_v)

    plsc.subcore_barrier()

    def _src(j):
        row0 = pl.multiple_of(w_off + j * _CHUNK, 8)
        return feat_hbm.at[pl.ds(row0, _CHUNK), pl.ds(col0, _CH)]

    sems = (sem0, sem1, sem2)

    pltpu.async_copy(_src(0), buf_v.at[0], sem0)
    pltpu.async_copy(_src(1), buf_v.at[1], sem1)

    iota16 = lax.iota(jnp.int32, _LANE)

    def _slot(t, k):
        j = 3 * t + k
        pltpu.make_async_copy(_src(j), buf_v.at[k], sems[k]).wait()
        buf = buf_v.at[k]

        ivs = [ids_v[j, pl.ds(g * _LANE, _LANE)] for g in range(8)]
        sf = ivs[0][0]
        sl = ivs[7][_LANE - 1]
        sfv = lax.broadcast(sf, (_LANE,))
        slv = lax.broadcast(sl, (_LANE,))
        nbv = plsc.all_reduce_population_count(ivs[0] == sfv)
        nlv = plsc.all_reduce_population_count(ivs[0] == slv)
        for g in range(1, 8):
            nbv = nbv + plsc.all_reduce_population_count(ivs[g] == sfv)
            nlv = nlv + plsc.all_reduce_population_count(ivs[g] == slv)
        nb = nbv[0]
        fast = jnp.logical_or((nb + nlv[0]) == _CHUNK, sf == sl)

        @pl.when(fast)
        def _():
            zeros8 = tuple(jnp.zeros((_LANE,), jnp.float32) for _ in range(8))

            def _rsum(i, accs):
                return tuple(
                    a + buf[i, pl.ds(kk * _LANE, _LANE)]
                    for kk, a in enumerate(accs)
                )

            run_a = lax.fori_loop(0, nb, _rsum, zeros8)
            run_b = lax.fori_loop(nb, _CHUNK, _rsum, zeros8)
            for kk in range(8):
                mini_v[0, pl.ds(kk * _LANE, _LANE)] = run_a[kk]
                mini_v[1, pl.ds(kk * _LANE, _LANE)] = run_b[kk]
            idx = jnp.where(
                iota16 == 0, sfv, jnp.where(iota16 == 1, slv, _N_SEG)
            )
            midx_v[0, pl.ds(0, _LANE)] = idx
            pltpu.sync_copy(mini_v, acc_sh.at[midx_v.at[0]], add=True)

        @pl.when(jnp.logical_not(fast))
        def _():
            pltpu.sync_copy(buf, acc_sh.at[ids_v.at[j]], add=True)

        @pl.when(j + 2 < _NCHUNK)
        def _():
            kn = (k + 2) % 3
            pltpu.async_copy(_src(j + 2), buf_v.at[kn], sems[kn])

    def _trio(t, carry):
        _slot(t, 0)
        _slot(t, 1)
        _slot(t, 2)
        return carry

    lax.fori_loop(0, (_NCHUNK - 1) // 3, _trio, 0)
    _slot((_NCHUNK - 1) // 3, 0)

    plsc.subcore_barrier()
    pltpu.sync_copy(
        acc_sh.at[pl.ds(s * _SEG_PER_TILE, _SEG_PER_TILE)],
        out_hbm.at[
            pl.ds(s * _SEG_PER_TILE, _SEG_PER_TILE), pl.ds(col0, _CH)
        ],
    )


_pool = pl.kernel(
    _pool_body,
    out_type=jax.ShapeDtypeStruct((_N_SEG, _D), jnp.float32),
    mesh=plsc.VectorSubcoreMesh(core_axis_name="c", subcore_axis_name="s"),
    scratch_types=[
        pltpu.VMEM((_NCHUNK, _CHUNK), jnp.int32),
        pltpu.VMEM((3, _CHUNK, _CH), jnp.float32),
        pltpu.VMEM((_ZROWS, _CH), jnp.float32),
        pltpu.VMEM((_LANE, _CH), jnp.float32),
        pltpu.VMEM((1, _LANE), jnp.int32),
        pltpu.VMEM_SHARED((_ACC_ROWS, _CH), jnp.float32),
        pltpu.SemaphoreType.DMA,
        pltpu.SemaphoreType.DMA,
        pltpu.SemaphoreType.DMA,
    ],
    compiler_params=pltpu.CompilerParams(needs_layout_passes=False),
)


@jax.jit
def kernel(feat, segment_ids):
    ids = segment_ids.astype(jnp.int32)
    win = jnp.stack(
        [lax.slice(ids, (int(o),), (int(o) + _WIN,)) for o in _W_OFF]
    )
    ids_padded = jnp.where(jnp.asarray(_OWN_MASK), win, _N_SEG).reshape(
        _NS, _NCHUNK, _CHUNK
    )
    return _pool(feat, ids_padded)
